# baseline (speedup 1.0000x reference)
import functools

import jax
import jax.numpy as jnp
from jax import lax
from jax.experimental import pallas as pl
from jax.experimental.pallas import tpu as pltpu
from jax.experimental.pallas import tpu_sc as plsc

_N_DIM = 128
_E_DIM = 64
_NLAYERS = 6
_E_SEL = 120
_NLOC = 8192
_NC = 2
_NS = 16
_NW = _NC * _NS
_NH = _NLOC // 2


def _silu(x):
    return x * (1.0 / (1.0 + jnp.exp(-x)))


def _sc_gather(table, idx, ch, tc_tiling=True):
    V, D = table.shape
    B = idx.shape[0]
    per_w = B // _NW
    nch = per_w // ch
    assert per_w % ch == 0 and B % _NW == 0
    mesh = plsc.VectorSubcoreMesh(core_axis_name="c", subcore_axis_name="s")
    params = (None if tc_tiling
              else pltpu.CompilerParams(use_tc_tiling_on_sc=False))

    @functools.partial(
        pl.kernel,
        out_type=jax.ShapeDtypeStruct((B, D), table.dtype),
        mesh=mesh,
        compiler_params=params,
        scratch_types=[
            pltpu.VMEM((per_w,), jnp.int32),
            pltpu.VMEM((ch, D), table.dtype),
            pltpu.VMEM((ch, D), table.dtype),
            pltpu.SemaphoreType.DMA,
            pltpu.SemaphoreType.DMA,
        ],
    )
    def k(table_hbm, idx_hbm, out_hbm, idx_all, buf0, buf1, s0, s1):
        wid = lax.axis_index("s") * _NC + lax.axis_index("c")
        base = wid * per_w
        pltpu.sync_copy(idx_hbm.at[pl.ds(base, per_w)], idx_all)
        bufs = ((buf0, s0), (buf1, s1))

        def issue(slot, chunk):
            bv, sem = slot
            pltpu.async_copy(table_hbm.at[idx_all.at[pl.ds(chunk * ch, ch)]],
                             bv, sem)

        issue(bufs[0], 0)

        @pl.loop(0, nch, step=2)
        def _body(c):
            for b in range(2):
                cc = c + b
                cur = bufs[b]
                nxt = bufs[1 - b]

                @pl.when(cc < nch)
                def _():
                    @pl.when(cc + 1 < nch)
                    def _():
                        issue(nxt, cc + 1)

                    bv, sem = cur
                    pltpu.make_async_copy(
                        table_hbm.at[idx_all.at[pl.ds(cc * ch, ch)]],
                        bv, sem).wait()
                    pltpu.sync_copy(bv, out_hbm.at[pl.ds(base + cc * ch, ch)])

    return k(table, idx)


def _sc_gather_xyzt(coord_flat, atype, idx, ch):
    B = idx.shape[0]
    per_w = B // _NW
    nch = per_w // ch
    assert per_w % ch == 0
    mesh = plsc.VectorSubcoreMesh(core_axis_name="c", subcore_axis_name="s")
    o1 = jax.ShapeDtypeStruct((B,), jnp.float32)
    oi = jax.ShapeDtypeStruct((B,), jnp.int32)

    @functools.partial(
        pl.kernel,
        out_type=(o1, o1, o1, oi),
        mesh=mesh,
        scratch_types=[
            pltpu.VMEM((4 * _NLOC,), jnp.float32),
            pltpu.VMEM((_NLOC,), jnp.int32),
            pltpu.VMEM((per_w,), jnp.int32),
            pltpu.VMEM((ch,), jnp.float32),
            pltpu.VMEM((ch,), jnp.float32),
            pltpu.VMEM((ch,), jnp.float32),
            pltpu.VMEM((ch,), jnp.int32),
        ],
        compiler_params=pltpu.CompilerParams(needs_layout_passes=False),
    )
    def k(tab_hbm, at_hbm, idx_hbm, x_hbm, y_hbm, z_hbm, t_hbm,
          tab_v, at_v, idx_all, x_v, y_v, z_v, t_v):
        wid = lax.axis_index("s") * _NC + lax.axis_index("c")
        base = wid * per_w
        pltpu.sync_copy(tab_hbm, tab_v)
        pltpu.sync_copy(at_hbm, at_v)
        pltpu.sync_copy(idx_hbm.at[pl.ds(base, per_w)], idx_all)

        @pl.loop(0, nch)
        def _chunk(c):
            off = base + c * ch

            @plsc.parallel_loop(0, ch // 16, unroll=8)
            def _grp(g):
                iv = idx_all[pl.ds(c * ch + g * 16, 16)]
                i4 = iv * 4
                x_v[pl.ds(g * 16, 16)] = plsc.load_gather(tab_v, [i4])
                y_v[pl.ds(g * 16, 16)] = plsc.load_gather(tab_v, [i4 + 1])
                z_v[pl.ds(g * 16, 16)] = plsc.load_gather(tab_v, [i4 + 2])
                t_v[pl.ds(g * 16, 16)] = plsc.load_gather(at_v, [iv])

            pltpu.sync_copy(x_v, x_hbm.at[pl.ds(off, ch)])
            pltpu.sync_copy(y_v, y_hbm.at[pl.ds(off, ch)])
            pltpu.sync_copy(z_v, z_hbm.at[pl.ds(off, ch)])
            pltpu.sync_copy(t_v, t_hbm.at[pl.ds(off, ch)])

    return k(coord_flat, atype, idx)


_BA = 64
_TC_PARAMS = pltpu.CompilerParams(dimension_semantics=("arbitrary",),
                                  vmem_limit_bytes=63 * 1024 * 1024)


def _geo_body(xe_ref, ye_ref, ze_ref, coord_ref, we_ref, be_ref,
              sw_ref, h0_ref, h1_ref, h2_ref, h3_ref, e0_ref):
    co = coord_ref[...]
    dx = xe_ref[...] - co[:, 0][:, None]
    dy = ye_ref[...] - co[:, 1][:, None]
    dz = ze_ref[...] - co[:, 2][:, None]
    r2 = dx * dx + dy * dy + dz * dz + 1e-12
    dist = jnp.sqrt(r2)
    uu = jnp.clip(dist - 5.0, 0.0, 1.0)
    sw = uu * uu * uu * (-6.0 * uu * uu + 15.0 * uu - 10.0) + 1.0
    inv = 1.0 / (dist + 1e-6)
    s1 = sw * inv
    s2 = s1 * inv
    h1 = s2 * dx
    h2 = s2 * dy
    h3 = s2 * dz
    sw_ref[...] = sw
    h0_ref[...] = s1
    h1_ref[...] = h1
    h2_ref[...] = h2
    h3_ref[...] = h3
    we = we_ref[...]
    t = (s1[:, :, None] * we[0][None, None, :]
         + h1[:, :, None] * we[1][None, None, :]
         + h2[:, :, None] * we[2][None, None, :]
         + h3[:, :, None] * we[3][None, None, :]
         + be_ref[...][None, :, :])
    e0_ref[...] = _silu(t)


def _layer_a0_body(edge_ref, tnb_ref, sw_ref, atype_ref, tt_ref,
                   weg_ref, wn_ref, ws_ref, wne_ref,
                   nodeo_ref, nw_ref):
    ba = atype_ref.shape[0]
    e2 = edge_ref[...].reshape(ba * _E_SEL, _E_DIM)
    gate = jnp.dot(e2, weg_ref[...], preferred_element_type=jnp.float32)
    gate = gate.reshape(ba, _E_SEL, _N_DIM)
    tt = tt_ref[...]
    ohn = (tnb_ref[...][:, :, None]
           == lax.broadcasted_iota(jnp.int32, (1, 1, 8), 2)).astype(jnp.float32)
    nnb = jnp.dot(ohn.reshape(ba * _E_SEL, 8), tt,
                  preferred_element_type=jnp.float32).reshape(ba, _E_SEL, _N_DIM)
    ohs = (atype_ref[...]
           == lax.broadcasted_iota(jnp.int32, (1, 8), 1)).astype(jnp.float32)
    node = jnp.dot(ohs, tt, preferred_element_type=jnp.float32)
    msg = gate * nnb * sw_ref[...][:, :, None]
    agg = jnp.sum(msg, axis=1) * (1.0 / 12.0)
    pre = (jnp.dot(agg, wn_ref[...], preferred_element_type=jnp.float32)
           + jnp.dot(node, ws_ref[...], preferred_element_type=jnp.float32))
    node_new = node + _silu(pre)
    nodeo_ref[...] = node_new
    nw_ref[...] = jnp.dot(node_new, wne_ref[...], preferred_element_type=jnp.float32)


def _layer_b_body(edge_ref, g_ref, sw_ref, nw_ref, wee_ref, edgeo_ref):
    ba = nw_ref.shape[0]
    e = edge_ref[...]
    e2 = e.reshape(ba * _E_SEL, _E_DIM)
    t = jnp.dot(e2, wee_ref[...], preferred_element_type=jnp.float32)
    t = (t.reshape(ba, _E_SEL, _E_DIM)
         + nw_ref[...][:, None, :]
         + g_ref[...][:, :, _E_DIM:])
    edgeo_ref[...] = e + _silu(t) * sw_ref[...][:, :, None]


def _layer_ba_body(edge_ref, g_ref, sw_ref, nw_ref, node_ref,
                   wee_ref, weg_ref, wn_ref, ws_ref, wne_ref,
                   edgeo_ref, nodeo_ref, nwo_ref):
    ba = nw_ref.shape[0]
    sw = sw_ref[...]
    g = g_ref[...]
    e = edge_ref[...]
    e2 = e.reshape(ba * _E_SEL, _E_DIM)
    t = jnp.dot(e2, wee_ref[...], preferred_element_type=jnp.float32)
    t = (t.reshape(ba, _E_SEL, _E_DIM)
         + nw_ref[...][:, None, :]
         + g[:, :, _E_DIM:])
    edge_new = e + _silu(t) * sw[:, :, None]
    edgeo_ref[...] = edge_new
    en2 = edge_new.reshape(ba * _E_SEL, _E_DIM)
    weg = weg_ref[...]
    gate_lo = jnp.dot(en2, weg[:, :_E_DIM],
                      preferred_element_type=jnp.float32).reshape(ba, _E_SEL, _E_DIM)
    gate_hi = jnp.dot(en2, weg[:, _E_DIM:],
                      preferred_element_type=jnp.float32).reshape(ba, _E_SEL, _E_DIM)
    wi = lax.bitcast_convert_type(g[:, :, :_E_DIM], jnp.int32)
    lo = lax.bitcast_convert_type(wi << 16, jnp.float32)
    hi = lax.bitcast_convert_type(wi & jnp.int32(-65536), jnp.float32)
    swn = sw[:, :, None] * (1.0 / 12.0)
    agg_lo = jnp.sum(gate_lo * lo * swn, axis=1)
    agg_hi = jnp.sum(gate_hi * hi * swn, axis=1)
    node = node_ref[...]
    wn = wn_ref[...]
    pre = (jnp.dot(agg_lo, wn[:_E_DIM, :], preferred_element_type=jnp.float32)
           + jnp.dot(agg_hi, wn[_E_DIM:, :], preferred_element_type=jnp.float32)
           + jnp.dot(node, ws_ref[...], preferred_element_type=jnp.float32))
    node_new = node + _silu(pre)
    nodeo_ref[...] = node_new
    nwo_ref[...] = jnp.dot(node_new, wne_ref[...],
                           preferred_element_type=jnp.float32)


def _final_body(h0_ref, h1_ref, h2_ref, h3_ref, edge_ref,
                c0_ref, c1_ref, c2_ref, c3_ref):
    e = edge_ref[...]
    scale = 1.0 / float(_E_SEL)
    gr0 = jnp.sum(h0_ref[...][:, :, None] * e, axis=1) * scale
    gr1 = jnp.sum(h1_ref[...][:, :, None] * e, axis=1) * scale
    gr2 = jnp.sum(h2_ref[...][:, :, None] * e, axis=1) * scale
    gr3 = jnp.sum(h3_ref[...][:, :, None] * e, axis=1) * scale
    grs = (gr0, gr1, gr2, gr3)
    crefs = (c0_ref, c1_ref, c2_ref, c3_ref)
    for a in range(4):
        col = (gr0 * gr0[:, a][:, None] + gr1 * gr1[:, a][:, None]
               + gr2 * gr2[:, a][:, None] + gr3 * gr3[:, a][:, None])
        crefs[a][...] = col


def _full(shape):
    nd = len(shape)
    return pl.BlockSpec(shape, lambda i: (0,) * nd)


def _blk(shape):
    nd = len(shape)
    return pl.BlockSpec(shape, lambda i: (i,) + (0,) * (nd - 1))


def _f32(shape):
    return jax.ShapeDtypeStruct(shape, jnp.float32)


def kernel(extended_coord, extended_atype, nlist, type_table, W_edge, b_edge,
           W_eg, W_node, W_self, W_ee, W_ne):
    coord = extended_coord[0]
    atype = extended_atype[0].astype(jnp.int32)
    nl = nlist[0].astype(jnp.int32)
    coordp = jnp.pad(coord, ((0, 0), (0, 1)))

    n_blocks = _NH // _BA
    idx_h = (nl[:_NH].reshape(-1), nl[_NH:].reshape(-1))
    atype2 = atype.reshape(_NLOC, 1)

    cg = [_sc_gather_xyzt(coordp.reshape(-1), atype, idx_h[h], ch=1024)
          for h in range(2)]
    xe = [cg[h][0].reshape(_NH, _E_SEL) for h in range(2)]
    ye = [cg[h][1].reshape(_NH, _E_SEL) for h in range(2)]
    ze = [cg[h][2].reshape(_NH, _E_SEL) for h in range(2)]
    tnb = [cg[h][3].reshape(_NH, _E_SEL) for h in range(2)]

    geo = pl.pallas_call(
        _geo_body,
        grid=(n_blocks,),
        in_specs=[
            _blk((_BA, _E_SEL)),
            _blk((_BA, _E_SEL)),
            _blk((_BA, _E_SEL)),
            _blk((_BA, 4)),
            _full((4, _E_DIM)),
            _full((1, _E_DIM)),
        ],
        out_specs=[
            _blk((_BA, _E_SEL)), _blk((_BA, _E_SEL)), _blk((_BA, _E_SEL)),
            _blk((_BA, _E_SEL)), _blk((_BA, _E_SEL)),
            _blk((_BA, _E_SEL, _E_DIM)),
        ],
        out_shape=[
            _f32((_NH, _E_SEL)), _f32((_NH, _E_SEL)), _f32((_NH, _E_SEL)),
            _f32((_NH, _E_SEL)), _f32((_NH, _E_SEL)),
            _f32((_NH, _E_SEL, _E_DIM)),
        ],
        compiler_params=_TC_PARAMS,
    )
    sw, h0, h1, h2, h3, edge = [list(t) for t in zip(
        geo(xe[0], ye[0], ze[0], coordp[:_NH], W_edge, b_edge.reshape(1, _E_DIM)),
        geo(xe[1], ye[1], ze[1], coordp[_NH:], W_edge, b_edge.reshape(1, _E_DIM)))]

    layer_a0 = pl.pallas_call(
        _layer_a0_body,
        grid=(n_blocks,),
        in_specs=[
            _blk((_BA, _E_SEL, _E_DIM)),
            _blk((_BA, _E_SEL)),
            _blk((_BA, _E_SEL)),
            _blk((_BA, 1)),
            _full((8, _N_DIM)),
            _full((_E_DIM, _N_DIM)),
            _full((_N_DIM, _N_DIM)),
            _full((_N_DIM, _N_DIM)),
            _full((_N_DIM, _E_DIM)),
        ],
        out_specs=[_blk((_BA, _N_DIM)), _blk((_BA, _E_DIM))],
        out_shape=[_f32((_NH, _N_DIM)), _f32((_NH, _E_DIM))],
        compiler_params=_TC_PARAMS,
    )

    layer_ba = pl.pallas_call(
        _layer_ba_body,
        grid=(n_blocks,),
        in_specs=[
            _blk((_BA, _E_SEL, _E_DIM)),
            _blk((_BA, _E_SEL, _N_DIM)),
            _blk((_BA, _E_SEL)),
            _blk((_BA, _E_DIM)),
            _blk((_BA, _N_DIM)),
            _full((_E_DIM, _E_DIM)),
            _full((_E_DIM, _N_DIM)),
            _full((_N_DIM, _N_DIM)),
            _full((_N_DIM, _N_DIM)),
            _full((_N_DIM, _E_DIM)),
        ],
        out_specs=[_blk((_BA, _E_SEL, _E_DIM)), _blk((_BA, _N_DIM)),
                   _blk((_BA, _E_DIM))],
        out_shape=[_f32((_NH, _E_SEL, _E_DIM)), _f32((_NH, _N_DIM)),
                   _f32((_NH, _E_DIM))],
        compiler_params=_TC_PARAMS,
    )

    layer_b = pl.pallas_call(
        _layer_b_body,
        grid=(n_blocks,),
        in_specs=[
            _blk((_BA, _E_SEL, _E_DIM)),
            _blk((_BA, _E_SEL, _N_DIM)),
            _blk((_BA, _E_SEL)),
            _blk((_BA, _E_DIM)),
            _full((_E_DIM, _E_DIM)),
        ],
        out_specs=_blk((_BA, _E_SEL, _E_DIM)),
        out_shape=_f32((_NH, _E_SEL, _E_DIM)),
        compiler_params=_TC_PARAMS,
    )

    def pack_table(node_new, nw_new):
        nb = node_new.astype(jnp.bfloat16)
        lo = lax.bitcast_convert_type(nb[:, :_E_DIM], jnp.uint16).astype(jnp.uint32)
        hi = lax.bitcast_convert_type(nb[:, _E_DIM:], jnp.uint16).astype(jnp.uint32)
        packed = lax.bitcast_convert_type(lo | (hi << 16), jnp.float32)
        return jnp.concatenate([packed, nw_new], axis=1)

    node, nw = [None, None], [None, None]
    for h in range(2):
        node[h], nw[h] = layer_a0(
            edge[h], tnb[h], sw[h], atype2[h * _NH:(h + 1) * _NH], type_table,
            W_eg[0], W_node[0], W_self[0], W_ne[0])
    tab = jnp.concatenate([pack_table(node[0], nw[0]),
                           pack_table(node[1], nw[1])], axis=0)
    for l in range(_NLAYERS):
        g = [_sc_gather(tab, idx_h[h], ch=256).reshape(_NH, _E_SEL, _N_DIM)
             for h in range(2)]
        if l + 1 < _NLAYERS:
            for h in range(2):
                edge[h], node[h], nw[h] = layer_ba(
                    edge[h], g[h], sw[h], nw[h], node[h],
                    W_ee[l], W_eg[l + 1], W_node[l + 1],
                    W_self[l + 1], W_ne[l + 1])
            tab = jnp.concatenate([pack_table(node[0], nw[0]),
                                   pack_table(node[1], nw[1])], axis=0)
        else:
            for h in range(2):
                edge[h] = layer_b(edge[h], g[h], sw[h], nw[h], W_ee[l])

    fin = pl.pallas_call(
        _final_body,
        grid=(n_blocks,),
        in_specs=[
            _blk((_BA, _E_SEL)), _blk((_BA, _E_SEL)), _blk((_BA, _E_SEL)),
            _blk((_BA, _E_SEL)),
            _blk((_BA, _E_SEL, _E_DIM)),
        ],
        out_specs=[_blk((_BA, _E_DIM))] * 4,
        out_shape=[_f32((_NH, _E_DIM))] * 4,
        compiler_params=_TC_PARAMS,
    )
    outs = []
    for h in range(2):
        c0, c1, c2, c3 = fin(h0[h], h1[h], h2[h], h3[h], edge[h])
        grrg = jnp.stack([c0, c1, c2, c3], axis=-1).reshape(_NH, _E_DIM * 4)
        outs.append(jnp.concatenate([node[h], grrg], axis=-1))
    return jnp.concatenate(outs, axis=0)[None]


# speedup vs baseline: 1.0207x; 1.0207x over previous
import functools

import jax
import jax.numpy as jnp
from jax import lax
from jax.experimental import pallas as pl
from jax.experimental.pallas import tpu as pltpu
from jax.experimental.pallas import tpu_sc as plsc

_N_DIM = 128
_E_DIM = 64
_NLAYERS = 6
_E_SEL = 120
_NLOC = 8192
_NC = 2
_NS = 16
_NW = _NC * _NS


def _silu(x):
    return x * (1.0 / (1.0 + jnp.exp(-x)))


def _sc_gather(table, idx, ch, tc_tiling=True):
    V, D = table.shape
    B = idx.shape[0]
    per_w = B // _NW
    nch = per_w // ch
    assert per_w % ch == 0 and B % _NW == 0
    mesh = plsc.VectorSubcoreMesh(core_axis_name="c", subcore_axis_name="s")
    params = (None if tc_tiling
              else pltpu.CompilerParams(use_tc_tiling_on_sc=False))

    @functools.partial(
        pl.kernel,
        out_type=jax.ShapeDtypeStruct((B, D), table.dtype),
        mesh=mesh,
        compiler_params=params,
        scratch_types=[
            pltpu.VMEM((per_w,), jnp.int32),
            pltpu.VMEM((ch, D), table.dtype),
            pltpu.VMEM((ch, D), table.dtype),
            pltpu.SemaphoreType.DMA,
            pltpu.SemaphoreType.DMA,
        ],
    )
    def k(table_hbm, idx_hbm, out_hbm, idx_all, buf0, buf1, s0, s1):
        wid = lax.axis_index("s") * _NC + lax.axis_index("c")
        base = wid * per_w
        pltpu.sync_copy(idx_hbm.at[pl.ds(base, per_w)], idx_all)
        bufs = ((buf0, s0), (buf1, s1))

        def issue(slot, chunk):
            bv, sem = slot
            pltpu.async_copy(table_hbm.at[idx_all.at[pl.ds(chunk * ch, ch)]],
                             bv, sem)

        issue(bufs[0], 0)

        @pl.loop(0, nch, step=2)
        def _body(c):
            for b in range(2):
                cc = c + b
                cur = bufs[b]
                nxt = bufs[1 - b]

                @pl.when(cc < nch)
                def _():
                    @pl.when(cc + 1 < nch)
                    def _():
                        issue(nxt, cc + 1)

                    bv, sem = cur
                    pltpu.make_async_copy(
                        table_hbm.at[idx_all.at[pl.ds(cc * ch, ch)]],
                        bv, sem).wait()
                    pltpu.sync_copy(bv, out_hbm.at[pl.ds(base + cc * ch, ch)])

    return k(table, idx)


def _sc_gather_xyzt(coord_flat, atype, idx, ch):
    B = idx.shape[0]
    per_w = B // _NW
    nch = per_w // ch
    assert per_w % ch == 0
    mesh = plsc.VectorSubcoreMesh(core_axis_name="c", subcore_axis_name="s")
    o1 = jax.ShapeDtypeStruct((B,), jnp.float32)
    oi = jax.ShapeDtypeStruct((B,), jnp.int32)

    @functools.partial(
        pl.kernel,
        out_type=(o1, o1, o1, oi),
        mesh=mesh,
        scratch_types=[
            pltpu.VMEM((4 * _NLOC,), jnp.float32),
            pltpu.VMEM((_NLOC,), jnp.int32),
            pltpu.VMEM((per_w,), jnp.int32),
            pltpu.VMEM((ch,), jnp.float32),
            pltpu.VMEM((ch,), jnp.float32),
            pltpu.VMEM((ch,), jnp.float32),
            pltpu.VMEM((ch,), jnp.int32),
        ],
        compiler_params=pltpu.CompilerParams(needs_layout_passes=False),
    )
    def k(tab_hbm, at_hbm, idx_hbm, x_hbm, y_hbm, z_hbm, t_hbm,
          tab_v, at_v, idx_all, x_v, y_v, z_v, t_v):
        wid = lax.axis_index("s") * _NC + lax.axis_index("c")
        base = wid * per_w
        pltpu.sync_copy(tab_hbm, tab_v)
        pltpu.sync_copy(at_hbm, at_v)
        pltpu.sync_copy(idx_hbm.at[pl.ds(base, per_w)], idx_all)

        @pl.loop(0, nch)
        def _chunk(c):
            off = base + c * ch

            @plsc.parallel_loop(0, ch // 16, unroll=8)
            def _grp(g):
                iv = idx_all[pl.ds(c * ch + g * 16, 16)]
                i4 = iv * 4
                x_v[pl.ds(g * 16, 16)] = plsc.load_gather(tab_v, [i4])
                y_v[pl.ds(g * 16, 16)] = plsc.load_gather(tab_v, [i4 + 1])
                z_v[pl.ds(g * 16, 16)] = plsc.load_gather(tab_v, [i4 + 2])
                t_v[pl.ds(g * 16, 16)] = plsc.load_gather(at_v, [iv])

            pltpu.sync_copy(x_v, x_hbm.at[pl.ds(off, ch)])
            pltpu.sync_copy(y_v, y_hbm.at[pl.ds(off, ch)])
            pltpu.sync_copy(z_v, z_hbm.at[pl.ds(off, ch)])
            pltpu.sync_copy(t_v, t_hbm.at[pl.ds(off, ch)])

    return k(coord_flat, atype, idx)


_BA = 64
_TC_PARAMS = pltpu.CompilerParams(dimension_semantics=("arbitrary",),
                                  vmem_limit_bytes=63 * 1024 * 1024)


def _geo_body(xe_ref, ye_ref, ze_ref, coord_ref, we_ref, be_ref,
              sw_ref, h0_ref, h1_ref, h2_ref, h3_ref, e0_ref):
    co = coord_ref[...]
    dx = xe_ref[...] - co[:, 0][:, None]
    dy = ye_ref[...] - co[:, 1][:, None]
    dz = ze_ref[...] - co[:, 2][:, None]
    r2 = dx * dx + dy * dy + dz * dz + 1e-12
    dist = jnp.sqrt(r2)
    uu = jnp.clip(dist - 5.0, 0.0, 1.0)
    sw = uu * uu * uu * (-6.0 * uu * uu + 15.0 * uu - 10.0) + 1.0
    inv = 1.0 / (dist + 1e-6)
    s1 = sw * inv
    s2 = s1 * inv
    h1 = s2 * dx
    h2 = s2 * dy
    h3 = s2 * dz
    sw_ref[...] = sw
    h0_ref[...] = s1
    h1_ref[...] = h1
    h2_ref[...] = h2
    h3_ref[...] = h3
    we = we_ref[...]
    t = (s1[:, :, None] * we[0][None, None, :]
         + h1[:, :, None] * we[1][None, None, :]
         + h2[:, :, None] * we[2][None, None, :]
         + h3[:, :, None] * we[3][None, None, :]
         + be_ref[...][None, :, :])
    e0_ref[...] = _silu(t)


def _layer_a0_body(edge_ref, tnb_ref, sw_ref, atype_ref, tt_ref,
                   weg_ref, wn_ref, ws_ref, wne_ref,
                   nodeo_ref, nw_ref):
    ba = atype_ref.shape[0]
    e2 = edge_ref[...].reshape(ba * _E_SEL, _E_DIM)
    gate = jnp.dot(e2, weg_ref[...], preferred_element_type=jnp.float32)
    gate = gate.reshape(ba, _E_SEL, _N_DIM)
    tt = tt_ref[...]
    ohn = (tnb_ref[...][:, :, None]
           == lax.broadcasted_iota(jnp.int32, (1, 1, 8), 2)).astype(jnp.float32)
    nnb = jnp.dot(ohn.reshape(ba * _E_SEL, 8), tt,
                  preferred_element_type=jnp.float32).reshape(ba, _E_SEL, _N_DIM)
    ohs = (atype_ref[...]
           == lax.broadcasted_iota(jnp.int32, (1, 8), 1)).astype(jnp.float32)
    node = jnp.dot(ohs, tt, preferred_element_type=jnp.float32)
    msg = gate * nnb * sw_ref[...][:, :, None]
    agg = jnp.sum(msg, axis=1) * (1.0 / 12.0)
    pre = (jnp.dot(agg, wn_ref[...], preferred_element_type=jnp.float32)
           + jnp.dot(node, ws_ref[...], preferred_element_type=jnp.float32))
    node_new = node + _silu(pre)
    nodeo_ref[...] = node_new
    nw_ref[...] = jnp.dot(node_new, wne_ref[...], preferred_element_type=jnp.float32)


def _layer_b_body(edge_ref, g_ref, sw_ref, nw_ref, wee_ref, edgeo_ref):
    ba = nw_ref.shape[0]
    e = edge_ref[...]
    e2 = e.reshape(ba * _E_SEL, _E_DIM)
    t = jnp.dot(e2, wee_ref[...], preferred_element_type=jnp.float32)
    t = (t.reshape(ba, _E_SEL, _E_DIM)
         + nw_ref[...][:, None, :]
         + g_ref[...][:, :, _E_DIM:])
    edgeo_ref[...] = e + _silu(t) * sw_ref[...][:, :, None]


def _layer_ba_body(edge_ref, g_ref, sw_ref, nw_ref, node_ref,
                   wee_ref, weg_ref, wn_ref, ws_ref, wne_ref,
                   edgeo_ref, nodeo_ref, nwo_ref):
    ba = nw_ref.shape[0]
    sw = sw_ref[...]
    g = g_ref[...]
    e = edge_ref[...]
    e2 = e.reshape(ba * _E_SEL, _E_DIM)
    t = jnp.dot(e2, wee_ref[...], preferred_element_type=jnp.float32)
    t = (t.reshape(ba, _E_SEL, _E_DIM)
         + nw_ref[...][:, None, :]
         + g[:, :, _E_DIM:])
    edge_new = e + _silu(t) * sw[:, :, None]
    edgeo_ref[...] = edge_new
    gate = jnp.dot(edge_new.reshape(ba * _E_SEL, _E_DIM), weg_ref[...],
                   preferred_element_type=jnp.float32)
    gate = gate.reshape(ba, _E_SEL, _N_DIM)
    wi = lax.bitcast_convert_type(g[:, :, :_E_DIM], jnp.int32)
    lo = lax.bitcast_convert_type(wi << 16, jnp.float32)
    hi = lax.bitcast_convert_type(wi & jnp.int32(-65536), jnp.float32)
    nnb = jnp.concatenate([lo, hi], axis=-1)
    msg = gate * nnb * sw[:, :, None]
    agg = jnp.sum(msg, axis=1) * (1.0 / 12.0)
    node = node_ref[...]
    pre = (jnp.dot(agg, wn_ref[...], preferred_element_type=jnp.float32)
           + jnp.dot(node, ws_ref[...], preferred_element_type=jnp.float32))
    node_new = node + _silu(pre)
    nodeo_ref[...] = node_new
    nwo_ref[...] = jnp.dot(node_new, wne_ref[...],
                           preferred_element_type=jnp.float32)


def _final_body(h0_ref, h1_ref, h2_ref, h3_ref, edge_ref,
                c0_ref, c1_ref, c2_ref, c3_ref):
    e = edge_ref[...]
    scale = 1.0 / float(_E_SEL)
    gr0 = jnp.sum(h0_ref[...][:, :, None] * e, axis=1) * scale
    gr1 = jnp.sum(h1_ref[...][:, :, None] * e, axis=1) * scale
    gr2 = jnp.sum(h2_ref[...][:, :, None] * e, axis=1) * scale
    gr3 = jnp.sum(h3_ref[...][:, :, None] * e, axis=1) * scale
    grs = (gr0, gr1, gr2, gr3)
    crefs = (c0_ref, c1_ref, c2_ref, c3_ref)
    for a in range(4):
        col = (gr0 * gr0[:, a][:, None] + gr1 * gr1[:, a][:, None]
               + gr2 * gr2[:, a][:, None] + gr3 * gr3[:, a][:, None])
        crefs[a][...] = col


def _full(shape):
    nd = len(shape)
    return pl.BlockSpec(shape, lambda i: (0,) * nd)


def _blk(shape):
    nd = len(shape)
    return pl.BlockSpec(shape, lambda i: (i,) + (0,) * (nd - 1))


def _f32(shape):
    return jax.ShapeDtypeStruct(shape, jnp.float32)


def kernel(extended_coord, extended_atype, nlist, type_table, W_edge, b_edge,
           W_eg, W_node, W_self, W_ee, W_ne):
    coord = extended_coord[0]
    atype = extended_atype[0].astype(jnp.int32)
    nl = nlist[0].astype(jnp.int32)
    idxe = nl.reshape(-1)
    coordp = jnp.pad(coord, ((0, 0), (0, 1)))

    n_blocks = _NLOC // _BA

    xe, ye, ze, tnb = _sc_gather_xyzt(coordp.reshape(-1), atype, idxe, ch=1024)
    xe = xe.reshape(_NLOC, _E_SEL)
    ye = ye.reshape(_NLOC, _E_SEL)
    ze = ze.reshape(_NLOC, _E_SEL)
    tnb = tnb.reshape(_NLOC, _E_SEL)

    geo = pl.pallas_call(
        _geo_body,
        grid=(n_blocks,),
        in_specs=[
            _blk((_BA, _E_SEL)),
            _blk((_BA, _E_SEL)),
            _blk((_BA, _E_SEL)),
            _blk((_BA, 4)),
            _full((4, _E_DIM)),
            _full((1, _E_DIM)),
        ],
        out_specs=[
            _blk((_BA, _E_SEL)), _blk((_BA, _E_SEL)), _blk((_BA, _E_SEL)),
            _blk((_BA, _E_SEL)), _blk((_BA, _E_SEL)),
            _blk((_BA, _E_SEL, _E_DIM)),
        ],
        out_shape=[
            _f32((_NLOC, _E_SEL)), _f32((_NLOC, _E_SEL)), _f32((_NLOC, _E_SEL)),
            _f32((_NLOC, _E_SEL)), _f32((_NLOC, _E_SEL)),
            _f32((_NLOC, _E_SEL, _E_DIM)),
        ],
        compiler_params=_TC_PARAMS,
    )
    sw, h0, h1, h2, h3, edge = geo(xe, ye, ze, coordp,
                                   W_edge, b_edge.reshape(1, _E_DIM))

    layer_a0 = pl.pallas_call(
        _layer_a0_body,
        grid=(n_blocks,),
        in_specs=[
            _blk((_BA, _E_SEL, _E_DIM)),
            _blk((_BA, _E_SEL)),
            _blk((_BA, _E_SEL)),
            _blk((_BA, 1)),
            _full((8, _N_DIM)),
            _full((_E_DIM, _N_DIM)),
            _full((_N_DIM, _N_DIM)),
            _full((_N_DIM, _N_DIM)),
            _full((_N_DIM, _E_DIM)),
        ],
        out_specs=[_blk((_BA, _N_DIM)), _blk((_BA, _E_DIM))],
        out_shape=[_f32((_NLOC, _N_DIM)), _f32((_NLOC, _E_DIM))],
        compiler_params=_TC_PARAMS,
    )

    layer_ba = pl.pallas_call(
        _layer_ba_body,
        grid=(n_blocks,),
        in_specs=[
            _blk((_BA, _E_SEL, _E_DIM)),
            _blk((_BA, _E_SEL, _N_DIM)),
            _blk((_BA, _E_SEL)),
            _blk((_BA, _E_DIM)),
            _blk((_BA, _N_DIM)),
            _full((_E_DIM, _E_DIM)),
            _full((_E_DIM, _N_DIM)),
            _full((_N_DIM, _N_DIM)),
            _full((_N_DIM, _N_DIM)),
            _full((_N_DIM, _E_DIM)),
        ],
        out_specs=[_blk((_BA, _E_SEL, _E_DIM)), _blk((_BA, _N_DIM)),
                   _blk((_BA, _E_DIM))],
        out_shape=[_f32((_NLOC, _E_SEL, _E_DIM)), _f32((_NLOC, _N_DIM)),
                   _f32((_NLOC, _E_DIM))],
        compiler_params=_TC_PARAMS,
    )

    layer_b = pl.pallas_call(
        _layer_b_body,
        grid=(n_blocks,),
        in_specs=[
            _blk((_BA, _E_SEL, _E_DIM)),
            _blk((_BA, _E_SEL, _N_DIM)),
            _blk((_BA, _E_SEL)),
            _blk((_BA, _E_DIM)),
            _full((_E_DIM, _E_DIM)),
        ],
        out_specs=_blk((_BA, _E_SEL, _E_DIM)),
        out_shape=_f32((_NLOC, _E_SEL, _E_DIM)),
        compiler_params=_TC_PARAMS,
    )

    def pack_table(node_new, nw_new):
        nb = node_new.astype(jnp.bfloat16)
        lo = lax.bitcast_convert_type(nb[:, :_E_DIM], jnp.uint16).astype(jnp.uint32)
        hi = lax.bitcast_convert_type(nb[:, _E_DIM:], jnp.uint16).astype(jnp.uint32)
        packed = lax.bitcast_convert_type(lo | (hi << 16), jnp.float32)
        return jnp.concatenate([packed, nw_new], axis=1)

    node, nw = layer_a0(edge, tnb, sw, atype.reshape(_NLOC, 1), type_table,
                        W_eg[0], W_node[0], W_self[0], W_ne[0])
    tab = pack_table(node, nw)
    for l in range(_NLAYERS):
        g = _sc_gather(tab, idxe, ch=256).reshape(_NLOC, _E_SEL, _N_DIM)
        if l + 1 < _NLAYERS:
            edge, node, nw = layer_ba(edge, g, sw, nw, node,
                                      W_ee[l], W_eg[l + 1], W_node[l + 1],
                                      W_self[l + 1], W_ne[l + 1])
            tab = pack_table(node, nw)
        else:
            edge = layer_b(edge, g, sw, nw, W_ee[l])

    fin = pl.pallas_call(
        _final_body,
        grid=(n_blocks,),
        in_specs=[
            _blk((_BA, _E_SEL)), _blk((_BA, _E_SEL)), _blk((_BA, _E_SEL)),
            _blk((_BA, _E_SEL)),
            _blk((_BA, _E_SEL, _E_DIM)),
        ],
        out_specs=[_blk((_BA, _E_DIM))] * 4,
        out_shape=[_f32((_NLOC, _E_DIM))] * 4,
        compiler_params=_TC_PARAMS,
    )
    c0, c1, c2, c3 = fin(h0, h1, h2, h3, edge)
    grrg = jnp.stack([c0, c1, c2, c3], axis=-1).reshape(_NLOC, _E_DIM * 4)
    out = jnp.concatenate([node, grrg], axis=-1)
    return out[None]
